# initial kernel scaffold (unmeasured)
import jax
import jax.numpy as jnp
from jax import lax
from jax.experimental import pallas as pl
from jax.experimental.pallas import tpu as pltpu

N_DEV = 4
M = 4096
K_SHARD = 1024
N = 8192
M_CHUNK = M // N_DEV
N_TILE = 1024


def kernel(x, w_mat):
    def body(x_ref, w_ref, out_ref, comm_ref, send_sems, recv_sems, out_sems):
        d = lax.axis_index("i")
        left = (d - 1) % N_DEV
        right = (d + 1) % N_DEV

        barrier_sem = pltpu.get_barrier_semaphore()
        for nbr in [left, right]:
            pl.semaphore_signal(
                barrier_sem, inc=1,
                device_id=(nbr,), device_id_type=pl.DeviceIdType.MESH,
            )
        pl.semaphore_wait(barrier_sem, 2)

        def accum_chunk(c, slot, first):
            xc = x_ref[pl.ds(c * M_CHUNK, M_CHUNK), :]
            for j in range(0, N, N_TILE):
                acc = jnp.dot(
                    xc, w_ref[:, j:j + N_TILE],
                    preferred_element_type=jnp.float32,
                )
                if not first:
                    acc = acc + comm_ref[slot, :, j:j + N_TILE].astype(
                        jnp.float32
                    )
                comm_ref[slot, :, j:j + N_TILE] = acc.astype(jnp.bfloat16)

        def write_out(slot, r, sem_idx):
            copy = pltpu.make_async_copy(
                comm_ref.at[slot],
                out_ref.at[pl.ds(r * M_CHUNK, M_CHUNK), :],
                out_sems.at[sem_idx],
            )
            copy.start()
            copy.wait()

        def hop(h):
            send_slot = h % 2
            recv_slot = (h + 1) % 2
            rdma = pltpu.make_async_remote_copy(
                src_ref=comm_ref.at[send_slot],
                dst_ref=comm_ref.at[recv_slot],
                send_sem=send_sems.at[h],
                recv_sem=recv_sems.at[h],
                device_id=(right,),
                device_id_type=pl.DeviceIdType.MESH,
            )
            rdma.start()
            rdma.wait()

        accum_chunk(d, 0, first=True)
        for s in range(N_DEV - 1):
            hop(s)
            accum_chunk((d - s - 1) % N_DEV, (s + 1) % 2, first=False)
        write_out(1, (d + 1) % N_DEV, 0)

        for t in range(N_DEV - 1):
            h = (N_DEV - 1) + t
            hop(h)
            recv_slot = (h + 1) % 2
            write_out(recv_slot, (d - t) % N_DEV, t + 1)

    out_shape = jax.ShapeDtypeStruct((M, N), jnp.bfloat16)
    return pl.pallas_call(
        body,
        out_shape=out_shape,
        in_specs=[
            pl.BlockSpec(memory_space=pltpu.VMEM),
            pl.BlockSpec(memory_space=pltpu.VMEM),
        ],
        out_specs=pl.BlockSpec(memory_space=pltpu.ANY),
        scratch_shapes=[
            pltpu.VMEM((2, M_CHUNK, N), jnp.bfloat16),
            pltpu.SemaphoreType.DMA((2 * (N_DEV - 1),)),
            pltpu.SemaphoreType.DMA((2 * (N_DEV - 1),)),
            pltpu.SemaphoreType.DMA((N_DEV,)),
        ],
        compiler_params=pltpu.CompilerParams(collective_id=0),
    )(x, w_mat)


# baseline (device time: 1266540 ns/iter reference)
import jax
import jax.numpy as jnp
from jax import lax
from jax.experimental import pallas as pl
from jax.experimental.pallas import tpu as pltpu

N_DEV = 4
M = 4096
K_SHARD = 1024
N = 8192
M_CHUNK = M // N_DEV
N_TILE = 1024


def kernel(x, w_mat):
    x = x.astype(jnp.bfloat16)
    w_mat = w_mat.astype(jnp.bfloat16)

    def body(x_ref, w_ref, out_ref, comm_ref, send_sems, recv_sems, out_sems):
        d = lax.axis_index("i")
        left = (d - 1) % N_DEV
        right = (d + 1) % N_DEV

        barrier_sem = pltpu.get_barrier_semaphore()
        for nbr in [left, right]:
            pl.semaphore_signal(
                barrier_sem, inc=1,
                device_id=(nbr,), device_id_type=pl.DeviceIdType.MESH,
            )
        pl.semaphore_wait(barrier_sem, 2)

        def accum_chunk(c, slot, first):
            xc = x_ref[pl.ds(c * M_CHUNK, M_CHUNK), :]
            for j in range(0, N, N_TILE):
                acc = jnp.dot(
                    xc, w_ref[:, j:j + N_TILE],
                    preferred_element_type=jnp.float32,
                )
                if not first:
                    acc = acc + comm_ref[slot, :, j:j + N_TILE].astype(
                        jnp.float32
                    )
                comm_ref[slot, :, j:j + N_TILE] = acc.astype(jnp.bfloat16)

        def write_out(slot, r, sem_idx):
            copy = pltpu.make_async_copy(
                comm_ref.at[slot],
                out_ref.at[pl.ds(r * M_CHUNK, M_CHUNK), :],
                out_sems.at[sem_idx],
            )
            copy.start()
            copy.wait()

        def hop(h):
            send_slot = h % 2
            recv_slot = (h + 1) % 2
            rdma = pltpu.make_async_remote_copy(
                src_ref=comm_ref.at[send_slot],
                dst_ref=comm_ref.at[recv_slot],
                send_sem=send_sems.at[h],
                recv_sem=recv_sems.at[h],
                device_id=(right,),
                device_id_type=pl.DeviceIdType.MESH,
            )
            rdma.start()
            rdma.wait()

        accum_chunk(d, 0, first=True)
        for s in range(N_DEV - 1):
            hop(s)
            accum_chunk((d - s - 1) % N_DEV, (s + 1) % 2, first=False)
        write_out(1, (d + 1) % N_DEV, 0)

        for t in range(N_DEV - 1):
            h = (N_DEV - 1) + t
            hop(h)
            recv_slot = (h + 1) % 2
            write_out(recv_slot, (d - t) % N_DEV, t + 1)

    out_shape = jax.ShapeDtypeStruct((M, N), jnp.bfloat16)
    return pl.pallas_call(
        body,
        out_shape=out_shape,
        in_specs=[
            pl.BlockSpec(memory_space=pltpu.MemorySpace.VMEM),
            pl.BlockSpec(memory_space=pltpu.MemorySpace.VMEM),
        ],
        out_specs=pl.BlockSpec(memory_space=pl.ANY),
        scratch_shapes=[
            pltpu.MemorySpace.VMEM((2, M_CHUNK, N), jnp.bfloat16),
            pltpu.SemaphoreType.DMA((2 * (N_DEV - 1),)),
            pltpu.SemaphoreType.DMA((2 * (N_DEV - 1),)),
            pltpu.SemaphoreType.DMA((N_DEV,)),
        ],
        compiler_params=pltpu.CompilerParams(
            collective_id=0,
            vmem_limit_bytes=100 * 1024 * 1024,
        ),
    )(x, w_mat)


# device time: 711130 ns/iter; 1.7810x vs baseline; 1.7810x over previous
import jax
import jax.numpy as jnp
from jax import lax
from jax.experimental import pallas as pl
from jax.experimental.pallas import tpu as pltpu

N_DEV = 4
M = 4096
K_SHARD = 1024
N = 8192
NH = N // 2
M_CHUNK = M // N_DEV
N_TILE = 1024
N_HOPS = 2 * (N_DEV - 1)


def kernel(x, w_mat):
    x = x.astype(jnp.bfloat16)
    w_mat = w_mat.astype(jnp.bfloat16)

    def body(x_ref, w_ref, out_ref, cR, cL, send_sems, recv_sems, out_sems):
        d = lax.axis_index("i")
        left = (d - 1) % N_DEV
        right = (d + 1) % N_DEV

        barrier_sem = pltpu.get_barrier_semaphore()
        for nbr in [left, right]:
            pl.semaphore_signal(
                barrier_sem, inc=1,
                device_id=(nbr,), device_id_type=pl.DeviceIdType.MESH,
            )
        pl.semaphore_wait(barrier_sem, 2)

        bufs = (cR, cL)
        col0s = (0, NH)
        tgts = (right, left)

        def accum(dirn, slot, c, first):
            buf, col0 = bufs[dirn], col0s[dirn]
            xc = x_ref[pl.ds(c * M_CHUNK, M_CHUNK), :]
            for j in range(0, NH, N_TILE):
                acc = jnp.dot(
                    xc, w_ref[:, col0 + j:col0 + j + N_TILE],
                    preferred_element_type=jnp.float32,
                )
                if not first:
                    acc = acc + buf[slot, :, j:j + N_TILE].astype(jnp.float32)
                buf[slot, :, j:j + N_TILE] = acc.astype(jnp.bfloat16)

        def make_rdma(dirn, h):
            buf = bufs[dirn]
            return pltpu.make_async_remote_copy(
                src_ref=buf.at[h % 2],
                dst_ref=buf.at[(h + 1) % 2],
                send_sem=send_sems.at[dirn, h],
                recv_sem=recv_sems.at[dirn, h],
                device_id=(tgts[dirn],),
                device_id_type=pl.DeviceIdType.MESH,
            )

        pending = {}

        def write_out(dirn, slot, r, idx):
            buf, col0 = bufs[dirn], col0s[dirn]
            copy = pltpu.make_async_copy(
                buf.at[slot],
                out_ref.at[pl.ds(r * M_CHUNK, M_CHUNK), col0:col0 + NH],
                out_sems.at[idx],
            )
            copy.start()
            pending[(dirn, slot)] = copy

        for dirn in (0, 1):
            accum(dirn, 0, d, first=True)

        for h in range(N_HOPS):
            recv_slot = (h + 1) % 2
            for dirn in (0, 1):
                cp = pending.pop((dirn, recv_slot), None)
                if cp is not None:
                    cp.wait()
            rdmas = [make_rdma(0, h), make_rdma(1, h)]
            for r_ in rdmas:
                r_.start()
            for dirn in (0, 1):
                rdmas[dirn].wait()
                if h < N_DEV - 1:
                    c = (d - h - 1) % N_DEV if dirn == 0 else (d + h + 1) % N_DEV
                    accum(dirn, recv_slot, c, first=False)
                    if h == N_DEV - 2:
                        rr = (d + 1) % N_DEV if dirn == 0 else (d - 1) % N_DEV
                        write_out(dirn, recv_slot, rr, 2 * (h - N_DEV + 2) + dirn)
                else:
                    t = h - (N_DEV - 1)
                    rr = (d - t) % N_DEV if dirn == 0 else (d + t) % N_DEV
                    write_out(dirn, recv_slot, rr, 2 * (h - N_DEV + 2) + dirn)

        for cp in pending.values():
            cp.wait()

    out_shape = jax.ShapeDtypeStruct((M, N), jnp.bfloat16)
    return pl.pallas_call(
        body,
        out_shape=out_shape,
        in_specs=[
            pl.BlockSpec(memory_space=pltpu.MemorySpace.VMEM),
            pl.BlockSpec(memory_space=pltpu.MemorySpace.VMEM),
        ],
        out_specs=pl.BlockSpec(memory_space=pl.ANY),
        scratch_shapes=[
            pltpu.MemorySpace.VMEM((2, M_CHUNK, NH), jnp.bfloat16),
            pltpu.MemorySpace.VMEM((2, M_CHUNK, NH), jnp.bfloat16),
            pltpu.SemaphoreType.DMA((2, N_HOPS)),
            pltpu.SemaphoreType.DMA((2, N_HOPS)),
            pltpu.SemaphoreType.DMA((2 * N_DEV,)),
        ],
        compiler_params=pltpu.CompilerParams(
            collective_id=0,
            vmem_limit_bytes=100 * 1024 * 1024,
        ),
    )(x, w_mat)


# device time: 637493 ns/iter; 1.9868x vs baseline; 1.1155x over previous
import jax
import jax.numpy as jnp
from jax import lax
from jax.experimental import pallas as pl
from jax.experimental.pallas import tpu as pltpu

N_DEV = 4
M = 4096
K_SHARD = 1024
N = 8192
NH = N // 2
M_CHUNK = M // N_DEV
N_TILE = 1024
N_HOPS = 2 * (N_DEV - 1)
N_SUB = 2
SUB = NH // N_SUB


def kernel(x, w_mat):
    x = x.astype(jnp.bfloat16)
    w_mat = w_mat.astype(jnp.bfloat16)

    def body(x_ref, w_ref, out_ref, cR, cL, send_sems, recv_sems, out_sems):
        d = lax.axis_index("i")
        left = (d - 1) % N_DEV
        right = (d + 1) % N_DEV

        barrier_sem = pltpu.get_barrier_semaphore()
        for nbr in [left, right]:
            pl.semaphore_signal(
                barrier_sem, inc=1,
                device_id=(nbr,), device_id_type=pl.DeviceIdType.MESH,
            )
        pl.semaphore_wait(barrier_sem, 2)

        bufs = (cR, cL)
        col0s = (0, NH)
        tgts = (right, left)

        def accum(dirn, slot, c, sub, first):
            buf, col0 = bufs[dirn], col0s[dirn]
            xc = x_ref[pl.ds(c * M_CHUNK, M_CHUNK), :]
            for j in range(sub * SUB, (sub + 1) * SUB, N_TILE):
                acc = jnp.dot(
                    xc, w_ref[:, col0 + j:col0 + j + N_TILE],
                    preferred_element_type=jnp.float32,
                )
                if not first:
                    acc = acc + buf[slot, :, j:j + N_TILE].astype(jnp.float32)
                buf[slot, :, j:j + N_TILE] = acc.astype(jnp.bfloat16)

        inflight = {}

        def start_hop(h, dirn, sub):
            buf = bufs[dirn]
            cols = pl.ds(sub * SUB, SUB)
            rdma = pltpu.make_async_remote_copy(
                src_ref=buf.at[h % 2, :, cols],
                dst_ref=buf.at[(h + 1) % 2, :, cols],
                send_sem=send_sems.at[dirn, h, sub],
                recv_sem=recv_sems.at[dirn, h, sub],
                device_id=(tgts[dirn],),
                device_id_type=pl.DeviceIdType.MESH,
            )
            rdma.start()
            inflight[(h, dirn, sub)] = rdma

        pending = {}

        def write_out(dirn, slot, r, idx):
            buf, col0 = bufs[dirn], col0s[dirn]
            copy = pltpu.make_async_copy(
                buf.at[slot],
                out_ref.at[pl.ds(r * M_CHUNK, M_CHUNK), col0:col0 + NH],
                out_sems.at[idx],
            )
            copy.start()
            pending[(dirn, slot)] = copy

        for sub in range(N_SUB):
            for dirn in (0, 1):
                accum(dirn, 0, d, sub, first=True)
                start_hop(0, dirn, sub)

        for h in range(N_HOPS):
            recv_slot = (h + 1) % 2
            for dirn in (0, 1):
                cp = pending.pop((dirn, recv_slot), None)
                if cp is not None:
                    cp.wait()
            for sub in range(N_SUB):
                for dirn in (0, 1):
                    inflight.pop((h, dirn, sub)).wait()
                    if h < N_DEV - 1:
                        c = (d - h - 1) % N_DEV if dirn == 0 else (d + h + 1) % N_DEV
                        accum(dirn, recv_slot, c, sub, first=False)
                    if h + 1 < N_HOPS:
                        start_hop(h + 1, dirn, sub)
            for dirn in (0, 1):
                if h == N_DEV - 2:
                    rr = (d + 1) % N_DEV if dirn == 0 else (d - 1) % N_DEV
                    write_out(dirn, recv_slot, rr, 2 * (h - N_DEV + 2) + dirn)
                elif h >= N_DEV - 1:
                    t = h - (N_DEV - 1)
                    rr = (d - t) % N_DEV if dirn == 0 else (d + t) % N_DEV
                    write_out(dirn, recv_slot, rr, 2 * (h - N_DEV + 2) + dirn)

        for cp in pending.values():
            cp.wait()

    out_shape = jax.ShapeDtypeStruct((M, N), jnp.bfloat16)
    return pl.pallas_call(
        body,
        out_shape=out_shape,
        in_specs=[
            pl.BlockSpec(memory_space=pltpu.MemorySpace.VMEM),
            pl.BlockSpec(memory_space=pltpu.MemorySpace.VMEM),
        ],
        out_specs=pl.BlockSpec(memory_space=pl.ANY),
        scratch_shapes=[
            pltpu.MemorySpace.VMEM((2, M_CHUNK, NH), jnp.bfloat16),
            pltpu.MemorySpace.VMEM((2, M_CHUNK, NH), jnp.bfloat16),
            pltpu.SemaphoreType.DMA((2, N_HOPS, N_SUB)),
            pltpu.SemaphoreType.DMA((2, N_HOPS, N_SUB)),
            pltpu.SemaphoreType.DMA((2 * N_DEV,)),
        ],
        compiler_params=pltpu.CompilerParams(
            collective_id=0,
            vmem_limit_bytes=100 * 1024 * 1024,
        ),
    )(x, w_mat)


# device time: 633294 ns/iter; 1.9999x vs baseline; 1.0066x over previous
import jax
import jax.numpy as jnp
from jax import lax
from jax.experimental import pallas as pl
from jax.experimental.pallas import tpu as pltpu

N_DEV = 4
M = 4096
K_SHARD = 1024
N = 8192
NH = N // 2
M_CHUNK = M // N_DEV
N_TILE = 1024
N_HOPS = 2 * (N_DEV - 1)
N_SUB = 4
SUB = NH // N_SUB


def kernel(x, w_mat):
    x = x.astype(jnp.bfloat16)
    w_mat = w_mat.astype(jnp.bfloat16)

    def body(x_ref, w_ref, out_ref, cR, cL, send_sems, recv_sems, out_sems):
        d = lax.axis_index("i")
        left = (d - 1) % N_DEV
        right = (d + 1) % N_DEV

        barrier_sem = pltpu.get_barrier_semaphore()
        for nbr in [left, right]:
            pl.semaphore_signal(
                barrier_sem, inc=1,
                device_id=(nbr,), device_id_type=pl.DeviceIdType.MESH,
            )
        pl.semaphore_wait(barrier_sem, 2)

        bufs = (cR, cL)
        col0s = (0, NH)
        tgts = (right, left)

        def accum(dirn, slot, c, sub, first):
            buf, col0 = bufs[dirn], col0s[dirn]
            xc = x_ref[pl.ds(c * M_CHUNK, M_CHUNK), :]
            for j in range(sub * SUB, (sub + 1) * SUB, N_TILE):
                acc = jnp.dot(
                    xc, w_ref[:, col0 + j:col0 + j + N_TILE],
                    preferred_element_type=jnp.float32,
                )
                if not first:
                    acc = acc + buf[slot, :, j:j + N_TILE].astype(jnp.float32)
                buf[slot, :, j:j + N_TILE] = acc.astype(jnp.bfloat16)

        inflight = {}

        def start_hop(h, dirn, sub):
            buf = bufs[dirn]
            cols = pl.ds(sub * SUB, SUB)
            rdma = pltpu.make_async_remote_copy(
                src_ref=buf.at[h % 2, :, cols],
                dst_ref=buf.at[(h + 1) % 2, :, cols],
                send_sem=send_sems.at[dirn, h, sub],
                recv_sem=recv_sems.at[dirn, h, sub],
                device_id=(tgts[dirn],),
                device_id_type=pl.DeviceIdType.MESH,
            )
            rdma.start()
            inflight[(h, dirn, sub)] = rdma

        pending = {}

        def write_out(dirn, slot, r, idx):
            buf, col0 = bufs[dirn], col0s[dirn]
            copy = pltpu.make_async_copy(
                buf.at[slot],
                out_ref.at[pl.ds(r * M_CHUNK, M_CHUNK), col0:col0 + NH],
                out_sems.at[idx],
            )
            copy.start()
            pending[(dirn, slot)] = copy

        for sub in range(N_SUB):
            for dirn in (0, 1):
                accum(dirn, 0, d, sub, first=True)
                start_hop(0, dirn, sub)

        for h in range(N_HOPS):
            recv_slot = (h + 1) % 2
            for dirn in (0, 1):
                cp = pending.pop((dirn, recv_slot), None)
                if cp is not None:
                    cp.wait()
            for sub in range(N_SUB):
                for dirn in (0, 1):
                    inflight.pop((h, dirn, sub)).wait()
                    if h < N_DEV - 1:
                        c = (d - h - 1) % N_DEV if dirn == 0 else (d + h + 1) % N_DEV
                        accum(dirn, recv_slot, c, sub, first=False)
                    if h + 1 < N_HOPS:
                        start_hop(h + 1, dirn, sub)
            for dirn in (0, 1):
                if h == N_DEV - 2:
                    rr = (d + 1) % N_DEV if dirn == 0 else (d - 1) % N_DEV
                    write_out(dirn, recv_slot, rr, 2 * (h - N_DEV + 2) + dirn)
                elif h >= N_DEV - 1:
                    t = h - (N_DEV - 1)
                    rr = (d - t) % N_DEV if dirn == 0 else (d + t) % N_DEV
                    write_out(dirn, recv_slot, rr, 2 * (h - N_DEV + 2) + dirn)

        for cp in pending.values():
            cp.wait()

    out_shape = jax.ShapeDtypeStruct((M, N), jnp.bfloat16)
    return pl.pallas_call(
        body,
        out_shape=out_shape,
        in_specs=[
            pl.BlockSpec(memory_space=pltpu.MemorySpace.VMEM),
            pl.BlockSpec(memory_space=pltpu.MemorySpace.VMEM),
        ],
        out_specs=pl.BlockSpec(memory_space=pl.ANY),
        scratch_shapes=[
            pltpu.MemorySpace.VMEM((2, M_CHUNK, NH), jnp.bfloat16),
            pltpu.MemorySpace.VMEM((2, M_CHUNK, NH), jnp.bfloat16),
            pltpu.SemaphoreType.DMA((2, N_HOPS, N_SUB)),
            pltpu.SemaphoreType.DMA((2, N_HOPS, N_SUB)),
            pltpu.SemaphoreType.DMA((2 * N_DEV,)),
        ],
        compiler_params=pltpu.CompilerParams(
            collective_id=0,
            vmem_limit_bytes=100 * 1024 * 1024,
        ),
    )(x, w_mat)


# device time: 601657 ns/iter; 2.1051x vs baseline; 1.0526x over previous
import jax
import jax.numpy as jnp
from jax import lax
from jax.experimental import pallas as pl
from jax.experimental.pallas import tpu as pltpu

N_DEV = 4
M = 4096
K_SHARD = 1024
N = 8192
NH = N // 2
M_CHUNK = M // N_DEV
N_TILE = 1024
N_HOPS = 2 * (N_DEV - 1)
N_SUB = 4
SUB = NH // N_SUB
N_WTILES = N // N_TILE


def kernel(x, w_mat):
    def body(x_ref, w_ref, out_ref, cR, cL, xb, wb, wstage, xstage,
             send_sems, recv_sems, out_sems, wstage_sems, xstage_sem):
        d = lax.axis_index("i")
        left = (d - 1) % N_DEV
        right = (d + 1) % N_DEV

        barrier_sem = pltpu.get_barrier_semaphore()
        for nbr in [left, right]:
            pl.semaphore_signal(
                barrier_sem, inc=1,
                device_id=(nbr,), device_id_type=pl.DeviceIdType.MESH,
            )
        pl.semaphore_wait(barrier_sem, 2)

        bufs = (cR, cL)
        col0s = (0, NH)
        tgts = (right, left)

        def w_tile_of_slot(k):
            sub, dirn = (k % 8) // 2, k % 2
            return 4 * dirn + sub

        def start_wload(k):
            tid = w_tile_of_slot(k)
            cp = pltpu.make_async_copy(
                w_ref.at[:, tid * N_TILE:(tid + 1) * N_TILE],
                wstage.at[k % 2],
                wstage_sems.at[k % 2],
            )
            cp.start()
            return cp

        wloads = {0: start_wload(0)}

        def wb_ready(k):
            if k + 1 < 4 * 8:
                wloads[k + 1] = start_wload(k + 1)
            wloads.pop(k).wait()
            wb[k % 2] = wstage[k % 2].astype(jnp.bfloat16)

        def convert_x(c):
            cp = pltpu.make_async_copy(
                x_ref.at[pl.ds(c * M_CHUNK, M_CHUNK), :],
                xstage, xstage_sem,
            )
            cp.start()
            cp.wait()
            xb[pl.ds(c * M_CHUNK, M_CHUNK), :] = xstage[...].astype(
                jnp.bfloat16
            )

        def accum(dirn, slot, c, sub, k, first):
            buf = bufs[dirn]
            wb_ready(k)
            xc = xb[pl.ds(c * M_CHUNK, M_CHUNK), :]
            j = sub * SUB
            acc = jnp.dot(xc, wb[k % 2], preferred_element_type=jnp.float32)
            if not first:
                acc = acc + buf[slot, :, j:j + SUB].astype(jnp.float32)
            buf[slot, :, j:j + SUB] = acc.astype(jnp.bfloat16)

        inflight = {}

        def start_hop(h, dirn, sub):
            buf = bufs[dirn]
            cols = pl.ds(sub * SUB, SUB)
            rdma = pltpu.make_async_remote_copy(
                src_ref=buf.at[h % 2, :, cols],
                dst_ref=buf.at[(h + 1) % 2, :, cols],
                send_sem=send_sems.at[dirn, h, sub],
                recv_sem=recv_sems.at[dirn, h, sub],
                device_id=(tgts[dirn],),
                device_id_type=pl.DeviceIdType.MESH,
            )
            rdma.start()
            inflight[(h, dirn, sub)] = rdma

        pending = {}

        def write_out(dirn, slot, r, idx):
            buf, col0 = bufs[dirn], col0s[dirn]
            copy = pltpu.make_async_copy(
                buf.at[slot],
                out_ref.at[pl.ds(r * M_CHUNK, M_CHUNK), col0:col0 + NH],
                out_sems.at[idx],
            )
            copy.start()
            pending[(dirn, slot)] = copy

        convert_x(d)
        k = 0
        for sub in range(N_SUB):
            for dirn in (0, 1):
                accum(dirn, 0, d, sub, k, first=True)
                start_hop(0, dirn, sub)
                k += 1
        convert_x((d - 1) % N_DEV)
        convert_x((d + 1) % N_DEV)

        for h in range(N_HOPS):
            recv_slot = (h + 1) % 2
            for dirn in (0, 1):
                cp = pending.pop((dirn, recv_slot), None)
                if cp is not None:
                    cp.wait()
            for sub in range(N_SUB):
                for dirn in (0, 1):
                    inflight.pop((h, dirn, sub)).wait()
                    if h < N_DEV - 1:
                        c = (d - h - 1) % N_DEV if dirn == 0 else (d + h + 1) % N_DEV
                        accum(dirn, recv_slot, c, sub, k, first=False)
                        k += 1
                    if h + 1 < N_HOPS:
                        start_hop(h + 1, dirn, sub)
                if h == 0 and sub == 0:
                    convert_x((d + 2) % N_DEV)
            for dirn in (0, 1):
                if h == N_DEV - 2:
                    rr = (d + 1) % N_DEV if dirn == 0 else (d - 1) % N_DEV
                    write_out(dirn, recv_slot, rr, 2 * (h - N_DEV + 2) + dirn)
                elif h >= N_DEV - 1:
                    t = h - (N_DEV - 1)
                    rr = (d - t) % N_DEV if dirn == 0 else (d + t) % N_DEV
                    write_out(dirn, recv_slot, rr, 2 * (h - N_DEV + 2) + dirn)

        for cp in pending.values():
            cp.wait()

    out_shape = jax.ShapeDtypeStruct((M, N), jnp.bfloat16)
    return pl.pallas_call(
        body,
        out_shape=out_shape,
        in_specs=[
            pl.BlockSpec(memory_space=pl.ANY),
            pl.BlockSpec(memory_space=pl.ANY),
        ],
        out_specs=pl.BlockSpec(memory_space=pl.ANY),
        scratch_shapes=[
            pltpu.MemorySpace.VMEM((2, M_CHUNK, NH), jnp.bfloat16),
            pltpu.MemorySpace.VMEM((2, M_CHUNK, NH), jnp.bfloat16),
            pltpu.MemorySpace.VMEM((M, K_SHARD), jnp.bfloat16),
            pltpu.MemorySpace.VMEM((2, K_SHARD, N_TILE), jnp.bfloat16),
            pltpu.MemorySpace.VMEM((2, K_SHARD, N_TILE), jnp.float32),
            pltpu.MemorySpace.VMEM((M_CHUNK, K_SHARD), jnp.float32),
            pltpu.SemaphoreType.DMA((2, N_HOPS, N_SUB)),
            pltpu.SemaphoreType.DMA((2, N_HOPS, N_SUB)),
            pltpu.SemaphoreType.DMA((2 * N_DEV,)),
            pltpu.SemaphoreType.DMA((2,)),
            pltpu.SemaphoreType.DMA,
        ],
        compiler_params=pltpu.CompilerParams(
            collective_id=0,
            vmem_limit_bytes=100 * 1024 * 1024,
        ),
    )(x, w_mat)
